# baseline (speedup 1.0000x reference)
import functools

import jax
import jax.numpy as jnp
from jax.experimental import pallas as pl
from jax.experimental.pallas import tpu as pltpu

N = 10000
D = 512
BM = 256
N_PAD = 10240


def _gcn_kernel(adj_ref, x_ref, w_ref, out_ref):
    t = jnp.dot(adj_ref[...], x_ref[...], preferred_element_type=jnp.float32)
    out_ref[...] = jnp.dot(t, w_ref[...], preferred_element_type=jnp.float32)


def kernel(x, adj, W):
    out = pl.pallas_call(
        _gcn_kernel,
        grid=(N_PAD // BM,),
        in_specs=[
            pl.BlockSpec((BM, N), lambda i: (i, 0)),
            pl.BlockSpec((N, D), lambda i: (0, 0)),
            pl.BlockSpec((D, D), lambda i: (0, 0)),
        ],
        out_specs=pl.BlockSpec((BM, D), lambda i: (i, 0)),
        out_shape=jax.ShapeDtypeStruct((N_PAD, D), jnp.float32),
        compiler_params=pltpu.CompilerParams(
            dimension_semantics=("parallel",),
            vmem_limit_bytes=100 * 1024 * 1024,
        ),
    )(adj, x, W)
    return out[:N]


# speedup vs baseline: 1.0745x; 1.0745x over previous
import functools

import jax
import jax.numpy as jnp
from jax.experimental import pallas as pl
from jax.experimental.pallas import tpu as pltpu

N = 10000
D = 512
BX = 2000
NX = N // BX
BM = 512
N_PAD = 10240
NM = N_PAD // BM


def _gcn_kernel(adj_ref, x_ref, w_ref, out_ref, s_ref):
    i = pl.program_id(0)

    @pl.when(i < NX)
    def _():
        sb = jnp.dot(x_ref[...], w_ref[...], preferred_element_type=jnp.float32)
        s_ref[pl.ds(i * BX, BX), :] = sb.astype(jnp.bfloat16)

    @pl.when(i >= NX)
    def _():
        a = adj_ref[...].astype(jnp.bfloat16)
        out_ref[...] = jnp.dot(a, s_ref[...], preferred_element_type=jnp.float32)


def kernel(x, adj, W):
    out = pl.pallas_call(
        _gcn_kernel,
        grid=(NX + NM,),
        in_specs=[
            pl.BlockSpec((BM, N), lambda i: (jnp.maximum(i - NX, 0), 0)),
            pl.BlockSpec((BX, D), lambda i: (jnp.minimum(i, NX - 1), 0)),
            pl.BlockSpec((D, D), lambda i: (0, 0)),
        ],
        out_specs=pl.BlockSpec((BM, D), lambda i: (jnp.maximum(i - NX, 0), 0)),
        out_shape=jax.ShapeDtypeStruct((N_PAD, D), jnp.float32),
        scratch_shapes=[pltpu.VMEM((N, D), jnp.bfloat16)],
        compiler_params=pltpu.CompilerParams(
            dimension_semantics=("arbitrary",),
            vmem_limit_bytes=100 * 1024 * 1024,
        ),
    )(adj, x, W)
    return out[:N]


# speedup vs baseline: 1.1969x; 1.1139x over previous
import functools

import jax
import jax.numpy as jnp
from jax.experimental import pallas as pl
from jax.experimental.pallas import tpu as pltpu

N = 10000
D = 512
BM = 512
NM = -(-N // BM)


def _gcn_kernel(adj_ref, x_ref, w_ref, out_ref):
    t = jnp.dot(adj_ref[...], x_ref[...], preferred_element_type=jnp.float32)
    out_ref[...] = jnp.dot(t, w_ref[...], preferred_element_type=jnp.float32)


def kernel(x, adj, W):
    return pl.pallas_call(
        _gcn_kernel,
        grid=(NM,),
        in_specs=[
            pl.BlockSpec((BM, N), lambda i: (i, 0)),
            pl.BlockSpec((N, D), lambda i: (0, 0)),
            pl.BlockSpec((D, D), lambda i: (0, 0)),
        ],
        out_specs=pl.BlockSpec((BM, D), lambda i: (i, 0)),
        out_shape=jax.ShapeDtypeStruct((N, D), jnp.float32),
        compiler_params=pltpu.CompilerParams(
            dimension_semantics=("parallel",),
            vmem_limit_bytes=100 * 1024 * 1024,
        ),
    )(adj, x, W)
